# Initial kernel scaffold; baseline (speedup 1.0000x reference)
import jax
import jax.numpy as jnp
from jax.experimental import pallas as pl


def kernel(input_ids, depths, sibling_indices, tree_attention_mask, word_embeddings, position_embeddings, token_type_embeddings, depth_embeddings, sibling_index_embeddings, ln_gamma, ln_beta):
    raise NotImplementedError("write your pallas kernel here")


# speedup vs baseline: 1.7509x; 1.7509x over previous
import functools

import jax
import jax.numpy as jnp
from jax import lax
from jax.experimental import pallas as pl
from jax.experimental.pallas import tpu as pltpu
from jax.experimental.pallas import tpu_sc as plsc

PAD = 1
EPS = 1e-5


def _posids_body(S, ids_full_ref, ids_blk_ref, out_ref):
    j = pl.program_id(0)
    CH = out_ref.shape[1]
    m_full = (ids_full_ref[...] != PAD).astype(jnp.bfloat16)
    r = lax.broadcasted_iota(jnp.int32, (S, CH), 0)
    c = lax.broadcasted_iota(jnp.int32, (S, CH), 1) + j * CH
    upper = (r <= c).astype(jnp.bfloat16)
    incr = lax.dot_general(
        m_full, upper, (((1,), (0,)), ((), ())),
        preferred_element_type=jnp.float32)
    mb = (ids_blk_ref[...] != PAD).astype(jnp.int32)
    out_ref[...] = incr.astype(jnp.int32) * mb + PAD


def _position_ids(input_ids):
    B, S = input_ids.shape
    CH = 256
    grid = (S // CH,)
    return pl.pallas_call(
        functools.partial(_posids_body, S),
        grid=grid,
        in_specs=[
            pl.BlockSpec((B, S), lambda j: (0, 0)),
            pl.BlockSpec((B, CH), lambda j: (0, j)),
        ],
        out_specs=pl.BlockSpec((B, CH), lambda j: (0, j)),
        out_shape=jax.ShapeDtypeStruct((B, S), jnp.int32),
    )(input_ids, input_ids)


_NW = 32
_CHUNK = 64


def _sc_gather(word_emb, pos_emb, idw, idp):
    n = idw.shape[0]
    hid = word_emb.shape[1]
    per = n // _NW
    nchunks = per // _CHUNK
    mesh = plsc.VectorSubcoreMesh(core_axis_name="c", subcore_axis_name="s")

    @functools.partial(
        pl.kernel,
        out_type=(
            jax.ShapeDtypeStruct((n, hid), jnp.float32),
            jax.ShapeDtypeStruct((n, hid), jnp.float32),
        ),
        mesh=mesh,
        scratch_types=[
            pltpu.VMEM((per,), jnp.int32),
            pltpu.VMEM((per,), jnp.int32),
            pltpu.VMEM((_CHUNK, hid), jnp.float32),
            pltpu.VMEM((_CHUNK, hid), jnp.float32),
            pltpu.SemaphoreType.DMA,
            pltpu.SemaphoreType.DMA,
        ],
    )
    def k(word_hbm, pos_hbm, idw_hbm, idp_hbm, wg_hbm, pg_hbm,
          idw_v, idp_v, bufw, bufp, semw, semp):
        wid = lax.axis_index("s") * 2 + lax.axis_index("c")
        base = wid * per
        pltpu.sync_copy(idw_hbm.at[pl.ds(base, per)], idw_v)
        pltpu.sync_copy(idp_hbm.at[pl.ds(base, per)], idp_v)
        for c in range(nchunks):
            cw = pltpu.async_copy(
                word_hbm.at[idw_v.at[pl.ds(c * _CHUNK, _CHUNK)]], bufw, semw)
            cp = pltpu.async_copy(
                pos_hbm.at[idp_v.at[pl.ds(c * _CHUNK, _CHUNK)]], bufp, semp)
            cw.wait()
            pltpu.sync_copy(bufw, wg_hbm.at[pl.ds(base + c * _CHUNK, _CHUNK)])
            cp.wait()
            pltpu.sync_copy(bufp, pg_hbm.at[pl.ds(base + c * _CHUNK, _CHUNK)])

    return k(word_emb, pos_emb, idw, idp)


def _combine_body(wg_ref, pg_ref, d_ref, s_ref, tm_ref,
                  dhi_ref, dlo_ref, shi_ref, slo_ref,
                  tt_ref, g_ref, b_ref, out_ref):
    T = out_ref.shape[0]
    nd = dhi_ref.shape[0]
    ns = shi_ref.shape[0]

    d_ids = d_ref[...]
    s_ids = s_ref[...]
    tm = tm_ref[...]
    dmask = (d_ids != -1).astype(jnp.float32) * tm
    smask = (s_ids != -1).astype(jnp.float32) * tm

    d_idx = jnp.clip(d_ids, 0, nd - 1)
    s_idx = jnp.clip(s_ids, 0, ns - 1)
    ohd = (lax.broadcasted_iota(jnp.int32, (T, nd), 1) == d_idx
           ).astype(jnp.bfloat16)
    ohs = (lax.broadcasted_iota(jnp.int32, (T, ns), 1) == s_idx
           ).astype(jnp.bfloat16)
    dn = (((1,), (0,)), ((), ()))
    demb = (lax.dot_general(ohd, dhi_ref[...], dn, preferred_element_type=jnp.float32)
            + lax.dot_general(ohd, dlo_ref[...], dn, preferred_element_type=jnp.float32))
    semb = (lax.dot_general(ohs, shi_ref[...], dn, preferred_element_type=jnp.float32)
            + lax.dot_general(ohs, slo_ref[...], dn, preferred_element_type=jnp.float32))

    x = (wg_ref[...] + pg_ref[...] + tt_ref[...]
         + demb * dmask + semb * smask)
    mu = jnp.mean(x, axis=-1, keepdims=True)
    xc = x - mu
    var = jnp.mean(xc * xc, axis=-1, keepdims=True)
    inv = lax.rsqrt(var + EPS)
    out_ref[...] = xc * inv * g_ref[...] + b_ref[...]


def _combine(wg, pg, depths_r, sibs_r, tmask_r, dhi, dlo, shi, slo,
             tt, gamma, beta):
    n, hid = wg.shape
    T = 256
    grid = (n // T,)
    nd = dhi.shape[0]
    ns = shi.shape[0]
    return pl.pallas_call(
        _combine_body,
        grid=grid,
        in_specs=[
            pl.BlockSpec((T, hid), lambda i: (i, 0)),
            pl.BlockSpec((T, hid), lambda i: (i, 0)),
            pl.BlockSpec((T, 1), lambda i: (i, 0)),
            pl.BlockSpec((T, 1), lambda i: (i, 0)),
            pl.BlockSpec((T, 1), lambda i: (i, 0)),
            pl.BlockSpec((nd, hid), lambda i: (0, 0)),
            pl.BlockSpec((nd, hid), lambda i: (0, 0)),
            pl.BlockSpec((ns, hid), lambda i: (0, 0)),
            pl.BlockSpec((ns, hid), lambda i: (0, 0)),
            pl.BlockSpec((1, hid), lambda i: (0, 0)),
            pl.BlockSpec((1, hid), lambda i: (0, 0)),
            pl.BlockSpec((1, hid), lambda i: (0, 0)),
        ],
        out_specs=pl.BlockSpec((T, hid), lambda i: (i, 0)),
        out_shape=jax.ShapeDtypeStruct((n, hid), jnp.float32),
    )(wg, pg, depths_r, sibs_r, tmask_r, dhi, dlo, shi, slo,
      tt, gamma, beta)


def kernel(input_ids, depths, sibling_indices, tree_attention_mask,
           word_embeddings, position_embeddings, token_type_embeddings,
           depth_embeddings, sibling_index_embeddings, ln_gamma, ln_beta):
    B, S = input_ids.shape
    n = B * S
    hid = word_embeddings.shape[1]

    input_ids = input_ids.astype(jnp.int32)
    position_ids = _position_ids(input_ids)

    wg, pg = _sc_gather(
        word_embeddings, position_embeddings,
        input_ids.reshape(n), position_ids.reshape(n))

    dhi = depth_embeddings.astype(jnp.bfloat16)
    dlo = (depth_embeddings - dhi.astype(jnp.float32)).astype(jnp.bfloat16)
    shi = sibling_index_embeddings.astype(jnp.bfloat16)
    slo = (sibling_index_embeddings - shi.astype(jnp.float32)
           ).astype(jnp.bfloat16)

    out = _combine(
        wg, pg,
        depths.reshape(n, 1).astype(jnp.int32),
        sibling_indices.reshape(n, 1).astype(jnp.int32),
        tree_attention_mask.reshape(n, 1).astype(jnp.float32),
        dhi, dlo, shi, slo,
        token_type_embeddings.astype(jnp.float32),
        ln_gamma.reshape(1, hid),
        ln_beta.reshape(1, hid),
    )
    return out.reshape(B, S, hid)


# speedup vs baseline: 1.9792x; 1.1304x over previous
import functools

import jax
import jax.numpy as jnp
from jax import lax
from jax.experimental import pallas as pl
from jax.experimental.pallas import tpu as pltpu
from jax.experimental.pallas import tpu_sc as plsc

PAD = 1
EPS = 1e-5


def _posids_body(S, ids_full_ref, ids_blk_ref, out_ref):
    j = pl.program_id(0)
    CH = out_ref.shape[1]
    m_full = (ids_full_ref[...] != PAD).astype(jnp.bfloat16)
    r = lax.broadcasted_iota(jnp.int32, (S, CH), 0)
    c = lax.broadcasted_iota(jnp.int32, (S, CH), 1) + j * CH
    upper = (r <= c).astype(jnp.bfloat16)
    incr = lax.dot_general(
        m_full, upper, (((1,), (0,)), ((), ())),
        preferred_element_type=jnp.float32)
    mb = (ids_blk_ref[...] != PAD).astype(jnp.int32)
    out_ref[...] = incr.astype(jnp.int32) * mb + PAD


def _position_ids(input_ids):
    B, S = input_ids.shape
    CH = 256
    grid = (S // CH,)
    return pl.pallas_call(
        functools.partial(_posids_body, S),
        grid=grid,
        in_specs=[
            pl.BlockSpec((B, S), lambda j: (0, 0)),
            pl.BlockSpec((B, CH), lambda j: (0, j)),
        ],
        out_specs=pl.BlockSpec((B, CH), lambda j: (0, j)),
        out_shape=jax.ShapeDtypeStruct((B, S), jnp.int32),
    )(input_ids, input_ids)


_NW = 32
_CHUNK = 32


def _sc_gather(word_emb, pos_emb, idw, idp):
    n = idw.shape[0]
    hid = word_emb.shape[1]
    per = n // _NW
    nchunks = per // _CHUNK
    mesh = plsc.VectorSubcoreMesh(core_axis_name="c", subcore_axis_name="s")

    @functools.partial(
        pl.kernel,
        out_type=(
            jax.ShapeDtypeStruct((n, hid), jnp.float32),
            jax.ShapeDtypeStruct((n, hid), jnp.float32),
        ),
        mesh=mesh,
        scratch_types=[
            pltpu.VMEM((per,), jnp.int32),
            pltpu.VMEM((per,), jnp.int32),
            pltpu.VMEM((_CHUNK, hid), jnp.float32),
            pltpu.VMEM((_CHUNK, hid), jnp.float32),
            pltpu.VMEM((_CHUNK, hid), jnp.float32),
            pltpu.VMEM((_CHUNK, hid), jnp.float32),
            pltpu.SemaphoreType.DMA,
            pltpu.SemaphoreType.DMA,
            pltpu.SemaphoreType.DMA,
            pltpu.SemaphoreType.DMA,
        ],
    )
    def k(word_hbm, pos_hbm, idw_hbm, idp_hbm, wg_hbm, pg_hbm,
          idw_v, idp_v, bufw0, bufw1, bufp0, bufp1,
          semgw, semgp, semsw, semsp):
        wid = lax.axis_index("s") * 2 + lax.axis_index("c")
        base = wid * per
        pltpu.sync_copy(idw_hbm.at[pl.ds(base, per)], idw_v)
        pltpu.sync_copy(idp_hbm.at[pl.ds(base, per)], idp_v)
        bufw = [bufw0, bufw1]
        bufp = [bufp0, bufp1]
        stw = [None, None]
        stp = [None, None]
        for c in range(nchunks):
            b = c & 1
            if c >= 2:
                stw[b].wait()
                stp[b].wait()
            gw = pltpu.async_copy(
                word_hbm.at[idw_v.at[pl.ds(c * _CHUNK, _CHUNK)]],
                bufw[b], semgw)
            gp = pltpu.async_copy(
                pos_hbm.at[idp_v.at[pl.ds(c * _CHUNK, _CHUNK)]],
                bufp[b], semgp)
            gw.wait()
            stw[b] = pltpu.async_copy(
                bufw[b], wg_hbm.at[pl.ds(base + c * _CHUNK, _CHUNK)], semsw)
            gp.wait()
            stp[b] = pltpu.async_copy(
                bufp[b], pg_hbm.at[pl.ds(base + c * _CHUNK, _CHUNK)], semsp)
        for h in (stw[0], stw[1], stp[0], stp[1]):
            if h is not None:
                h.wait()

    return k(word_emb, pos_emb, idw, idp)


_T = 256


def _combine_body(wg_ref, pg_ref, d_ref, s_ref, tm_ref,
                  dhi_ref, dlo_ref, shi_ref, slo_ref,
                  tt_ref, g_ref, b_ref, out_ref):
    T = out_ref.shape[0]
    nd = dhi_ref.shape[0]
    ns = shi_ref.shape[0]

    ids_d = d_ref[0, 0, :]
    ids_s = s_ref[0, 0, :]
    tm = tm_ref[0, 0, :]

    dscale = ((ids_d != -1).astype(jnp.float32) * tm).astype(jnp.bfloat16)
    sscale = ((ids_s != -1).astype(jnp.float32) * tm).astype(jnp.bfloat16)
    d_idx = jnp.clip(ids_d, 0, nd - 1)
    s_idx = jnp.clip(ids_s, 0, ns - 1)
    ohd = ((lax.broadcasted_iota(jnp.int32, (nd, T), 0) == d_idx[None, :]
            ).astype(jnp.bfloat16)) * dscale[None, :]
    ohs = ((lax.broadcasted_iota(jnp.int32, (ns, T), 0) == s_idx[None, :]
            ).astype(jnp.bfloat16)) * sscale[None, :]
    dn = (((0,), (0,)), ((), ()))
    demb = (lax.dot_general(ohd, dhi_ref[...], dn, preferred_element_type=jnp.float32)
            + lax.dot_general(ohd, dlo_ref[...], dn, preferred_element_type=jnp.float32))
    semb = (lax.dot_general(ohs, shi_ref[...], dn, preferred_element_type=jnp.float32)
            + lax.dot_general(ohs, slo_ref[...], dn, preferred_element_type=jnp.float32))

    x = wg_ref[...] + pg_ref[...] + tt_ref[...] + demb + semb
    mu = jnp.mean(x, axis=-1, keepdims=True)
    xc = x - mu
    var = jnp.mean(xc * xc, axis=-1, keepdims=True)
    inv = lax.rsqrt(var + EPS)
    out_ref[...] = xc * inv * g_ref[...] + b_ref[...]


def _combine_half(prev, wg, pg, d3, s3, tm3, dhi, dlo, shi, slo,
                  tt, gamma, beta, half, n_total):
    nh, hid = wg.shape
    nblk = nh // _T
    off = half * nblk
    nd = dhi.shape[0]
    ns = shi.shape[0]
    in_specs = [
        pl.BlockSpec((_T, hid), lambda i: (i, 0)),
        pl.BlockSpec((_T, hid), lambda i: (i, 0)),
        pl.BlockSpec((1, 1, _T), lambda i: (i + off, 0, 0)),
        pl.BlockSpec((1, 1, _T), lambda i: (i + off, 0, 0)),
        pl.BlockSpec((1, 1, _T), lambda i: (i + off, 0, 0)),
        pl.BlockSpec((nd, hid), lambda i: (0, 0)),
        pl.BlockSpec((nd, hid), lambda i: (0, 0)),
        pl.BlockSpec((ns, hid), lambda i: (0, 0)),
        pl.BlockSpec((ns, hid), lambda i: (0, 0)),
        pl.BlockSpec((1, hid), lambda i: (0, 0)),
        pl.BlockSpec((1, hid), lambda i: (0, 0)),
        pl.BlockSpec((1, hid), lambda i: (0, 0)),
    ]
    args = [wg, pg, d3, s3, tm3, dhi, dlo, shi, slo, tt, gamma, beta]
    aliases = {}
    if prev is not None:
        in_specs = [pl.BlockSpec(memory_space=pl.ANY)] + in_specs
        args = [prev] + args
        aliases = {0: 0}
    if prev is None:
        def body2(*refs):
            _combine_body(*refs)
    else:
        def body2(prev_ref, *refs):
            _combine_body(*refs)
    return pl.pallas_call(
        body2,
        grid=(nblk,),
        in_specs=in_specs,
        out_specs=pl.BlockSpec((_T, hid), lambda i: (i + off, 0)),
        out_shape=jax.ShapeDtypeStruct((n_total, hid), jnp.float32),
        input_output_aliases=aliases,
    )(*args)


def kernel(input_ids, depths, sibling_indices, tree_attention_mask,
           word_embeddings, position_embeddings, token_type_embeddings,
           depth_embeddings, sibling_index_embeddings, ln_gamma, ln_beta):
    B, S = input_ids.shape
    n = B * S
    nh = n // 2
    hid = word_embeddings.shape[1]

    input_ids = input_ids.astype(jnp.int32)
    position_ids = _position_ids(input_ids)

    idw = input_ids.reshape(n)
    idp = position_ids.reshape(n)

    dhi = depth_embeddings.astype(jnp.bfloat16)
    dlo = (depth_embeddings - dhi.astype(jnp.float32)).astype(jnp.bfloat16)
    shi = sibling_index_embeddings.astype(jnp.bfloat16)
    slo = (sibling_index_embeddings - shi.astype(jnp.float32)
           ).astype(jnp.bfloat16)

    nblk = n // _T
    d3 = depths.reshape(nblk, 1, _T).astype(jnp.int32)
    s3 = sibling_indices.reshape(nblk, 1, _T).astype(jnp.int32)
    tm3 = tree_attention_mask.reshape(nblk, 1, _T).astype(jnp.float32)
    tt = token_type_embeddings.astype(jnp.float32)
    g2 = ln_gamma.reshape(1, hid)
    b2 = ln_beta.reshape(1, hid)

    out = None
    for half in range(2):
        sl = slice(half * nh, (half + 1) * nh)
        wg, pg = _sc_gather(
            word_embeddings, position_embeddings, idw[sl], idp[sl])
        out = _combine_half(out, wg, pg, d3, s3, tm3, dhi, dlo, shi, slo,
                            tt, g2, b2, half, n)
    return out.reshape(B, S, hid)
